# baseline (speedup 1.0000x reference)
import functools

import jax
import jax.numpy as jnp
from jax import lax
from jax.experimental import pallas as pl
from jax.experimental.pallas import tpu as pltpu
from jax.experimental.pallas import tpu_sc as plsc

DIM = 32
SEQ = 50
NUM_ROWS = 16384
NUM_CORES = 2
NUM_SUBCORES = 16
NW = NUM_CORES * NUM_SUBCORES
ROWS_PER_W = NUM_ROWS // NW
RCHUNK = 32
CHUNK = RCHUNK * SEQ
GGRP = 4
NCHUNK = ROWS_PER_W // RCHUNK


@functools.partial(
    pl.kernel,
    mesh=plsc.VectorSubcoreMesh(core_axis_name="c", subcore_axis_name="s"),
    out_type=jax.ShapeDtypeStruct((NUM_ROWS, SEQ, DIM), jnp.float32),
    scratch_types=[
        pltpu.VMEM((CHUNK,), jnp.int32),
        pltpu.VMEM((CHUNK,), jnp.int32),
        pltpu.VMEM((CHUNK, DIM), jnp.float32),
        pltpu.VMEM((CHUNK, DIM), jnp.float32),
        pltpu.SemaphoreType.DMA,
        pltpu.SemaphoreType.DMA,
        pltpu.SemaphoreType.DMA,
        pltpu.SemaphoreType.DMA,
        pltpu.SemaphoreType.DMA,
        pltpu.SemaphoreType.DMA,
    ],
    compiler_params=pltpu.CompilerParams(use_tc_tiling_on_sc=False),
)
def _sc_gather(x_hbm, table_hbm, out_hbm,
               idx0, idx1, rows0, rows1,
               isem0, isem1, gsem0, gsem1, osem0, osem1):
    idx_v = (idx0, idx1)
    rows_v = (rows0, rows1)
    isem = (isem0, isem1)
    gsem = (gsem0, gsem1)
    osem = (osem0, osem1)

    wid = lax.axis_index("s") * NUM_CORES + lax.axis_index("c")
    ibase = wid * ROWS_PER_W * SEQ
    rbase = wid * ROWS_PER_W

    def idx_load(g):
        s = g % 2
        return pltpu.async_copy(x_hbm.at[pl.ds(ibase + g * CHUNK, CHUNK)],
                                idx_v[s], isem[s])

    def gather_start(g):
        s = g % 2
        w = GGRP * SEQ
        return [pltpu.async_copy(table_hbm.at[idx_v[s].at[pl.ds(q * w, w)]],
                                 rows_v[s].at[pl.ds(q * w, w)], gsem[s])
                for q in range(RCHUNK // GGRP)]

    def store_start(g):
        s = g % 2
        return [pltpu.async_copy(rows_v[s].at[pl.ds(r * SEQ, SEQ)],
                                 out_hbm.at[rbase + g * RCHUNK + r], osem[s])
                for r in range(RCHUNK)]

    iloads = {0: idx_load(0), 1: idx_load(1)}
    gathers = {}
    stores = {}
    iloads[0].wait()
    gathers[0] = gather_start(0)
    for g in range(NCHUNK):
        if g + 1 < NCHUNK:
            iloads[g + 1].wait()
            if g - 1 >= 0:
                for h in stores[g - 1]:
                    h.wait()
            gathers[g + 1] = gather_start(g + 1)
        for h in gathers[g]:
            h.wait()
        if g + 2 < NCHUNK:
            iloads[g + 2] = idx_load(g + 2)
        stores[g] = store_start(g)
    for h in stores[NCHUNK - 2]:
        h.wait()
    for h in stores[NCHUNK - 1]:
        h.wait()


def kernel(x, E):
    return _sc_gather(x.reshape(-1), E)
